# baseline (speedup 1.0000x reference)
import functools

import jax
import jax.numpy as jnp
from jax import lax
from jax.experimental import pallas as pl
from jax.experimental.pallas import tpu as pltpu
from jax.experimental.pallas import tpu_sc as plsc

V = 100000
H = 128
T = 200
B = 1024
G3 = 3 * H

_SPLITS = (24, 88, 88)
_TBASE = tuple(sum(_SPLITS[:i]) for i in range(len(_SPLITS)))

_NC, _NS = 2, 16
_NW = _NC * _NS
_CHUNK = 128


def _sc_gather(emb, idx1d):
    n = idx1d.shape[0]
    cpw = n // (_NW * _CHUNK)
    rpw = cpw * _CHUNK
    mesh = plsc.VectorSubcoreMesh(core_axis_name="c", subcore_axis_name="s")

    @functools.partial(
        pl.kernel,
        mesh=mesh,
        out_type=jax.ShapeDtypeStruct((n, H), jnp.float32),
        scratch_types=[
            pltpu.VMEM((rpw,), jnp.int32),
            pltpu.VMEM((_CHUNK, H), jnp.float32),
            pltpu.VMEM((_CHUNK, H), jnp.float32),
            pltpu.SemaphoreType.DMA,
            pltpu.SemaphoreType.DMA,
        ],
    )
    def k(emb_hbm, idx_hbm, out_hbm, idx_v, buf0, buf1, sem0, sem1):
        wid = lax.axis_index("s") * _NC + lax.axis_index("c")
        c0 = wid * cpw
        pltpu.sync_copy(idx_hbm.at[pl.ds(c0 * _CHUNK, rpw)], idx_v)

        def start(j, buf, sem):
            pltpu.make_async_copy(
                emb_hbm.at[idx_v.at[pl.ds(j * _CHUNK, _CHUNK)]], buf, sem
            ).start()

        def finish(j, buf, sem):
            pltpu.make_async_copy(
                emb_hbm.at[idx_v.at[pl.ds(j * _CHUNK, _CHUNK)]], buf, sem
            ).wait()
            pltpu.sync_copy(buf, out_hbm.at[pl.ds((c0 + j) * _CHUNK, _CHUNK)])

        start(0, buf0, sem0)

        def body(g, carry):
            j0 = 2 * g
            start(j0 + 1, buf1, sem1)
            finish(j0, buf0, sem0)

            @pl.when(j0 + 2 < cpw)
            def _():
                start(j0 + 2, buf0, sem0)

            finish(j0 + 1, buf1, sem1)
            return carry

        lax.fori_loop(0, cpw // 2, body, 0)
        if cpw % 2:
            finish(cpw - 1, buf0, sem0)

    return k(emb, idx1d)


_UNROLL = 8


def _gru_layer_step(x_bf, h, wrz, win, whn, brz, bin_, bhn, mask):
    hb = h.astype(jnp.bfloat16)
    xh = jnp.concatenate([x_bf, hb], axis=1)
    grz = jnp.dot(xh, wrz[...], preferred_element_type=jnp.float32) + brz[...]
    gin = jnp.dot(x_bf, win[...], preferred_element_type=jnp.float32) + bin_[...]
    ghn = jnp.dot(hb, whn[...], preferred_element_type=jnp.float32) + bhn[...]
    u_r = jnp.tanh(grz[:, 0:H])
    u_z = jnp.tanh(grz[:, H:])
    n = jnp.tanh(gin + ghn * (u_r + 1.0))
    hn = 0.5 * ((h + n) + u_z * (h - n))
    h_next = jnp.where(mask, hn, h)
    out = jnp.where(mask, hn, 0.0)
    return h_next, out


def _make_gru_chunk_kernel(t_base, t_len):
    def body(lens_ref, wrz0, win0, whn0, brz0, bin0, bhn0,
             wrz1, win1, whn1, brz1, bin1, bhn1,
             h0_in, h1_in, x_ref, *refs):
        if len(refs) == 6:
            refs = refs[1:]
        o1_ref, h0_out, h1_out, h0_ref, h1_ref = refs
        t = pl.program_id(0)

        @pl.when(t == 0)
        def _():
            h0_ref[...] = h0_in[...]
            h1_ref[...] = h1_in[...]

        h0n = h0_ref[...]
        h1n = h1_ref[...]
        for k in range(_UNROLL):
            tk = t_base + t * _UNROLL + k
            mask = lens_ref[...] > tk
            x_t = x_ref[k].astype(jnp.bfloat16)
            h0n, o0 = _gru_layer_step(x_t, h0n, wrz0, win0, whn0,
                                      brz0, bin0, bhn0, mask)
            h1n, o1 = _gru_layer_step(o0.astype(jnp.bfloat16), h1n,
                                      wrz1, win1, whn1, brz1, bin1, bhn1, mask)
            o1_ref[k] = o1
        h0_ref[...] = h0n
        h1_ref[...] = h1n

        @pl.when(t == t_len // _UNROLL - 1)
        def _():
            h0_out[...] = h0n
            h1_out[...] = h1n

    return body


def _gru_chunk(chunk, x_c, lens2d, h0_in, h1_in, o1_buf, weights):
    full = lambda shape: pl.BlockSpec(shape, lambda t: (0,) * len(shape))
    wspecs = [full((2 * H, 2 * H)), full((H, H)), full((H, H)),
              full((1, 2 * H)), full((1, H)), full((1, H))]
    t_base, t_len = _TBASE[chunk], _SPLITS[chunk]
    blk0 = t_base // _UNROLL
    alias_in = [pl.BlockSpec(memory_space=pl.ANY)] if chunk else []
    alias_arg = (o1_buf,) if chunk else ()
    return pl.pallas_call(
        _make_gru_chunk_kernel(t_base, t_len),
        grid=(t_len // _UNROLL,),
        in_specs=[full((B, 1))] + wspecs + wspecs + [
            full((B, H)), full((B, H)),
            pl.BlockSpec((_UNROLL, B, H), lambda t: (t, 0, 0)),
        ] + alias_in,
        out_specs=[
            pl.BlockSpec((_UNROLL, B, H), lambda t, _b=blk0: (t + _b, 0, 0)),
            full((B, H)), full((B, H)),
        ],
        out_shape=[
            jax.ShapeDtypeStruct((T, B, H), jnp.float32),
            jax.ShapeDtypeStruct((B, H), jnp.float32),
            jax.ShapeDtypeStruct((B, H), jnp.float32),
        ],
        scratch_shapes=[
            pltpu.VMEM((B, H), jnp.float32),
            pltpu.VMEM((B, H), jnp.float32),
        ],
        input_output_aliases={16: 0} if chunk else {},
        compiler_params=pltpu.CompilerParams(
            dimension_semantics=("arbitrary",),
        ),
    )(lens2d, *weights, h0_in, h1_in, x_c, *alias_arg)


def _prep_layer(W_ih, W_hh, b_ih, b_hh):
    bf = jnp.bfloat16
    wi, wh = W_ih.T, W_hh.T
    wrz = (0.5 * jnp.concatenate([wi[:, 0:2 * H], wh[:, 0:2 * H]],
                                 axis=0)).astype(bf)
    win = wi[:, 2 * H:].astype(bf)
    whn = (0.5 * wh[:, 2 * H:]).astype(bf)
    brz = (0.5 * (b_ih[0:2 * H] + b_hh[0:2 * H])).reshape(1, 2 * H)
    bin_ = b_ih[2 * H:].reshape(1, H)
    bhn = (0.5 * b_hh[2 * H:]).reshape(1, H)
    return wrz, win, whn, brz, bin_, bhn


def kernel(inputs, inputsLen, emb, W_ih0, W_hh0, b_ih0, b_hh0,
           W_ih1, W_hh1, b_ih1, b_hh1):
    idx = inputs.astype(jnp.int32)
    lens2d = inputsLen.astype(jnp.int32).reshape(B, 1)
    weights = (*_prep_layer(W_ih0, W_hh0, b_ih0, b_hh0),
               *_prep_layer(W_ih1, W_hh1, b_ih1, b_hh1))

    xs = [
        _sc_gather(emb, idx[t0:t0 + tl].reshape(tl * B)).reshape(tl, B, H)
        for t0, tl in zip(_TBASE, _SPLITS)
    ]

    h0 = jnp.zeros((B, H), jnp.float32)
    h1 = jnp.zeros((B, H), jnp.float32)
    o1 = None
    for c in range(len(_SPLITS)):
        o1, h0, h1 = _gru_chunk(c, xs[c], lens2d, h0, h1, o1, weights)

    hidden = jnp.stack([h0, h1], axis=0)
    return o1, hidden


# speedup vs baseline: 1.0024x; 1.0024x over previous
import functools

import jax
import jax.numpy as jnp
from jax import lax
from jax.experimental import pallas as pl
from jax.experimental.pallas import tpu as pltpu
from jax.experimental.pallas import tpu_sc as plsc

V = 100000
H = 128
T = 200
B = 1024

_SPLITS = (24, 88, 88)
_TBASE = tuple(sum(_SPLITS[:i]) for i in range(len(_SPLITS)))

_NC, _NS = 2, 16
_NW = _NC * _NS
_CHUNK = 128


def _sc_gather(emb, idx1d):
    n = idx1d.shape[0]
    cpw = n // (_NW * _CHUNK)
    rpw = cpw * _CHUNK
    mesh = plsc.VectorSubcoreMesh(core_axis_name="c", subcore_axis_name="s")

    @functools.partial(
        pl.kernel,
        mesh=mesh,
        out_type=jax.ShapeDtypeStruct((n, H), jnp.float32),
        scratch_types=[
            pltpu.VMEM((rpw,), jnp.int32),
            pltpu.VMEM((_CHUNK, H), jnp.float32),
            pltpu.VMEM((_CHUNK, H), jnp.float32),
            pltpu.SemaphoreType.DMA,
            pltpu.SemaphoreType.DMA,
        ],
    )
    def k(emb_hbm, idx_hbm, out_hbm, idx_v, buf0, buf1, sem0, sem1):
        wid = lax.axis_index("s") * _NC + lax.axis_index("c")
        c0 = wid * cpw
        pltpu.sync_copy(idx_hbm.at[pl.ds(c0 * _CHUNK, rpw)], idx_v)

        def start(j, buf, sem):
            pltpu.make_async_copy(
                emb_hbm.at[idx_v.at[pl.ds(j * _CHUNK, _CHUNK)]], buf, sem
            ).start()

        def finish(j, buf, sem):
            pltpu.make_async_copy(
                emb_hbm.at[idx_v.at[pl.ds(j * _CHUNK, _CHUNK)]], buf, sem
            ).wait()
            pltpu.sync_copy(buf, out_hbm.at[pl.ds((c0 + j) * _CHUNK, _CHUNK)])

        start(0, buf0, sem0)

        def body(g, carry):
            j0 = 2 * g
            start(j0 + 1, buf1, sem1)
            finish(j0, buf0, sem0)

            @pl.when(j0 + 2 < cpw)
            def _():
                start(j0 + 2, buf0, sem0)

            finish(j0 + 1, buf1, sem1)
            return carry

        lax.fori_loop(0, cpw // 2, body, 0)
        if cpw % 2:
            finish(cpw - 1, buf0, sem0)

    return k(emb, idx1d)


_UNROLL = 8


def _gru_layer_step(x_bf, h, wrz, win, whn, brz, bin_, bhn, mask):
    hb = h.astype(jnp.bfloat16)
    xh = jnp.concatenate([x_bf, hb], axis=1)
    grz = jnp.dot(xh, wrz[...], preferred_element_type=jnp.float32) + brz[...]
    gin = jnp.dot(x_bf, win[...], preferred_element_type=jnp.float32) + bin_[...]
    ghn = jnp.dot(hb, whn[...], preferred_element_type=jnp.float32) + bhn[...]
    u_r = jnp.tanh(grz[:, 0:H])
    u_z = jnp.tanh(grz[:, H:])
    n = jnp.tanh(gin + ghn * (u_r + 1.0))
    hn = 0.5 * ((h + n) + u_z * (h - n))
    h_next = jnp.where(mask, hn, h)
    out = jnp.where(mask, hn, 0.0)
    return h_next, out


def _make_gru_chunk_kernel(t_base, t_len):
    def body(lens_ref, wrz0, win0, whn0, brz0, bin0, bhn0,
             wrz1, win1, whn1, brz1, bin1, bhn1,
             h0_in, h1_in, x_ref, *refs):
        if len(refs) == 6:
            refs = refs[1:]
        o1_ref, h0_out, h1_out, h0_ref, h1_ref = refs
        t = pl.program_id(0)

        @pl.when(t == 0)
        def _():
            h0_ref[...] = h0_in[...]
            h1_ref[...] = h1_in[...]

        h0n = h0_ref[...]
        h1n = h1_ref[...]
        for k in range(_UNROLL):
            tk = t_base + t * _UNROLL + k
            mask = lens_ref[...] > tk
            x_t = x_ref[k].astype(jnp.bfloat16)
            h0n, o0 = _gru_layer_step(x_t, h0n, wrz0, win0, whn0,
                                      brz0, bin0, bhn0, mask)
            h1n, o1 = _gru_layer_step(o0.astype(jnp.bfloat16), h1n,
                                      wrz1, win1, whn1, brz1, bin1, bhn1, mask)
            o1_ref[k] = o1
        h0_ref[...] = h0n
        h1_ref[...] = h1n

        @pl.when(t == t_len // _UNROLL - 1)
        def _():
            h0_out[...] = h0n
            h1_out[...] = h1n

    return body


def _gru_chunk(chunk, x_c, lens2d, h0_in, h1_in, o1_buf, weights):
    full = lambda shape: pl.BlockSpec(shape, lambda t: (0,) * len(shape))
    wspecs = [full((2 * H, 2 * H)), full((H, H)), full((H, H)),
              full((1, 2 * H)), full((1, H)), full((1, H))]
    t_base, t_len = _TBASE[chunk], _SPLITS[chunk]
    blk0 = t_base // _UNROLL
    alias_in = [pl.BlockSpec(memory_space=pl.ANY)] if chunk else []
    alias_arg = (o1_buf,) if chunk else ()
    return pl.pallas_call(
        _make_gru_chunk_kernel(t_base, t_len),
        grid=(t_len // _UNROLL,),
        in_specs=[full((B, 1))] + wspecs + wspecs + [
            full((B, H)), full((B, H)),
            pl.BlockSpec((_UNROLL, B, H), lambda t: (t, 0, 0)),
        ] + alias_in,
        out_specs=[
            pl.BlockSpec((_UNROLL, B, H), lambda t, _b=blk0: (t + _b, 0, 0)),
            full((B, H)), full((B, H)),
        ],
        out_shape=[
            jax.ShapeDtypeStruct((T, B, H), jnp.float32),
            jax.ShapeDtypeStruct((B, H), jnp.float32),
            jax.ShapeDtypeStruct((B, H), jnp.float32),
        ],
        scratch_shapes=[
            pltpu.VMEM((B, H), jnp.float32),
            pltpu.VMEM((B, H), jnp.float32),
        ],
        input_output_aliases={16: 0} if chunk else {},
        compiler_params=pltpu.CompilerParams(
            dimension_semantics=("arbitrary",),
        ),
    )(lens2d, *weights, h0_in, h1_in, x_c, *alias_arg)


def _prep_layer(W_ih, W_hh, b_ih, b_hh):
    bf = jnp.bfloat16
    wi, wh = W_ih.T, W_hh.T
    wrz = (0.5 * jnp.concatenate([wi[:, 0:2 * H], wh[:, 0:2 * H]],
                                 axis=0)).astype(bf)
    win = wi[:, 2 * H:].astype(bf)
    whn = (0.5 * wh[:, 2 * H:]).astype(bf)
    brz = (0.5 * (b_ih[0:2 * H] + b_hh[0:2 * H])).reshape(1, 2 * H)
    bin_ = b_ih[2 * H:].reshape(1, H)
    bhn = (0.5 * b_hh[2 * H:]).reshape(1, H)
    return wrz, win, whn, brz, bin_, bhn


def kernel(inputs, inputsLen, emb, W_ih0, W_hh0, b_ih0, b_hh0,
           W_ih1, W_hh1, b_ih1, b_hh1):
    idx = inputs.astype(jnp.int32)
    lens2d = inputsLen.astype(jnp.int32).reshape(B, 1)
    weights = (*_prep_layer(W_ih0, W_hh0, b_ih0, b_hh0),
               *_prep_layer(W_ih1, W_hh1, b_ih1, b_hh1))

    xs = [
        _sc_gather(emb, idx[t0:t0 + tl].reshape(tl * B)).reshape(tl, B, H)
        for t0, tl in zip(_TBASE, _SPLITS)
    ]

    h0 = jnp.zeros((B, H), jnp.float32)
    h1 = jnp.zeros((B, H), jnp.float32)
    o1 = None
    for c in range(len(_SPLITS)):
        o1, h0, h1 = _gru_chunk(c, xs[c], lens2d, h0, h1, o1, weights)

    hidden = jnp.stack([h0, h1], axis=0)
    return o1, hidden


# speedup vs baseline: 1.0057x; 1.0033x over previous
import functools

import jax
import jax.numpy as jnp
from jax import lax
from jax.experimental import pallas as pl
from jax.experimental.pallas import tpu as pltpu
from jax.experimental.pallas import tpu_sc as plsc

V = 100000
H = 128
T = 200
B = 1024

_SPLITS = (24, 88, 88)
_TBASE = tuple(sum(_SPLITS[:i]) for i in range(len(_SPLITS)))

_NC, _NS = 2, 16
_NW = _NC * _NS
_CHUNK = 128


def _sc_gather(emb, idx1d):
    n = idx1d.shape[0]
    cpw = n // (_NW * _CHUNK)
    rpw = cpw * _CHUNK
    mesh = plsc.VectorSubcoreMesh(core_axis_name="c", subcore_axis_name="s")

    @functools.partial(
        pl.kernel,
        mesh=mesh,
        out_type=jax.ShapeDtypeStruct((n, H), jnp.float32),
        scratch_types=[
            pltpu.VMEM((rpw,), jnp.int32),
            pltpu.VMEM((_CHUNK, H), jnp.float32),
            pltpu.VMEM((_CHUNK, H), jnp.float32),
            pltpu.SemaphoreType.DMA,
            pltpu.SemaphoreType.DMA,
        ],
    )
    def k(emb_hbm, idx_hbm, out_hbm, idx_v, buf0, buf1, sem0, sem1):
        wid = lax.axis_index("s") * _NC + lax.axis_index("c")
        c0 = wid * cpw
        pltpu.sync_copy(idx_hbm.at[pl.ds(c0 * _CHUNK, rpw)], idx_v)

        def start(j, buf, sem):
            pltpu.make_async_copy(
                emb_hbm.at[idx_v.at[pl.ds(j * _CHUNK, _CHUNK)]], buf, sem
            ).start()

        def finish(j, buf, sem):
            pltpu.make_async_copy(
                emb_hbm.at[idx_v.at[pl.ds(j * _CHUNK, _CHUNK)]], buf, sem
            ).wait()
            pltpu.sync_copy(buf, out_hbm.at[pl.ds((c0 + j) * _CHUNK, _CHUNK)])

        start(0, buf0, sem0)

        def body(g, carry):
            j0 = 2 * g
            start(j0 + 1, buf1, sem1)
            finish(j0, buf0, sem0)

            @pl.when(j0 + 2 < cpw)
            def _():
                start(j0 + 2, buf0, sem0)

            finish(j0 + 1, buf1, sem1)
            return carry

        lax.fori_loop(0, cpw // 2, body, 0)
        if cpw % 2:
            finish(cpw - 1, buf0, sem0)

    return k(emb, idx1d)


_UNROLL = 8


def _gru_layer_step(x_bf, h, wrz, win, whn, brz, bin_, bhn, mask):
    hb = h.astype(jnp.bfloat16)
    xh = jnp.concatenate([x_bf, hb], axis=1)
    grz = jnp.dot(xh, wrz[...], preferred_element_type=jnp.float32) + brz[...]
    gin = jnp.dot(x_bf, win[...], preferred_element_type=jnp.float32) + bin_[...]
    ghn = jnp.dot(hb, whn[...], preferred_element_type=jnp.float32) + bhn[...]
    u_r = jnp.tanh(grz[:, 0:H])
    u_z = jnp.tanh(grz[:, H:])
    n = jnp.tanh(gin + ghn * (u_r + 1.0))
    hn = 0.5 * ((h + n) + u_z * (h - n))
    h_next = jnp.where(mask, hn, h)
    out = jnp.where(mask, hn, 0.0)
    return h_next, out


def _make_gru_chunk_kernel(t_base, t_len):
    def body(lens_ref, wrz0, win0, whn0, brz0, bin0, bhn0,
             wrz1, win1, whn1, brz1, bin1, bhn1,
             h0_in, h1_in, x_ref, *refs):
        if len(refs) == 4:
            refs = refs[1:]
        o1_ref, h0_out, h1_out = refs
        t = pl.program_id(0)

        first = t == 0
        h0n = jnp.where(first, h0_in[...], h0_out[...])
        h1n = jnp.where(first, h1_in[...], h1_out[...])
        for k in range(_UNROLL):
            tk = t_base + t * _UNROLL + k
            mask = lens_ref[...] > tk
            x_t = x_ref[k].astype(jnp.bfloat16)
            h0n, o0 = _gru_layer_step(x_t, h0n, wrz0, win0, whn0,
                                      brz0, bin0, bhn0, mask)
            h1n, o1 = _gru_layer_step(o0.astype(jnp.bfloat16), h1n,
                                      wrz1, win1, whn1, brz1, bin1, bhn1, mask)
            o1_ref[k] = o1
        h0_out[...] = h0n
        h1_out[...] = h1n

    return body


def _gru_chunk(chunk, x_c, lens2d, h0_in, h1_in, o1_buf, weights):
    full = lambda shape: pl.BlockSpec(shape, lambda t: (0,) * len(shape))
    wspecs = [full((2 * H, 2 * H)), full((H, H)), full((H, H)),
              full((1, 2 * H)), full((1, H)), full((1, H))]
    t_base, t_len = _TBASE[chunk], _SPLITS[chunk]
    blk0 = t_base // _UNROLL
    alias_in = [pl.BlockSpec(memory_space=pl.ANY)] if chunk else []
    alias_arg = (o1_buf,) if chunk else ()
    return pl.pallas_call(
        _make_gru_chunk_kernel(t_base, t_len),
        grid=(t_len // _UNROLL,),
        in_specs=[full((B, 1))] + wspecs + wspecs + [
            full((B, H)), full((B, H)),
            pl.BlockSpec((_UNROLL, B, H), lambda t: (t, 0, 0)),
        ] + alias_in,
        out_specs=[
            pl.BlockSpec((_UNROLL, B, H), lambda t, _b=blk0: (t + _b, 0, 0)),
            full((B, H)), full((B, H)),
        ],
        out_shape=[
            jax.ShapeDtypeStruct((T, B, H), jnp.float32),
            jax.ShapeDtypeStruct((B, H), jnp.float32),
            jax.ShapeDtypeStruct((B, H), jnp.float32),
        ],
        input_output_aliases={16: 0} if chunk else {},
        compiler_params=pltpu.CompilerParams(
            dimension_semantics=("arbitrary",),
        ),
    )(lens2d, *weights, h0_in, h1_in, x_c, *alias_arg)


def _prep_layer(W_ih, W_hh, b_ih, b_hh):
    bf = jnp.bfloat16
    wi, wh = W_ih.T, W_hh.T
    wrz = (0.5 * jnp.concatenate([wi[:, 0:2 * H], wh[:, 0:2 * H]],
                                 axis=0)).astype(bf)
    win = wi[:, 2 * H:].astype(bf)
    whn = (0.5 * wh[:, 2 * H:]).astype(bf)
    brz = (0.5 * (b_ih[0:2 * H] + b_hh[0:2 * H])).reshape(1, 2 * H)
    bin_ = b_ih[2 * H:].reshape(1, H)
    bhn = (0.5 * b_hh[2 * H:]).reshape(1, H)
    return wrz, win, whn, brz, bin_, bhn


def kernel(inputs, inputsLen, emb, W_ih0, W_hh0, b_ih0, b_hh0,
           W_ih1, W_hh1, b_ih1, b_hh1):
    idx = inputs.astype(jnp.int32)
    lens2d = inputsLen.astype(jnp.int32).reshape(B, 1)
    weights = (*_prep_layer(W_ih0, W_hh0, b_ih0, b_hh0),
               *_prep_layer(W_ih1, W_hh1, b_ih1, b_hh1))

    xs = [
        _sc_gather(emb, idx[t0:t0 + tl].reshape(tl * B)).reshape(tl, B, H)
        for t0, tl in zip(_TBASE, _SPLITS)
    ]

    h0 = jnp.zeros((B, H), jnp.float32)
    h1 = jnp.zeros((B, H), jnp.float32)
    o1 = None
    for c in range(len(_SPLITS)):
        o1, h0, h1 = _gru_chunk(c, xs[c], lens2d, h0, h1, o1, weights)

    hidden = jnp.stack([h0, h1], axis=0)
    return o1, hidden
